# baseline (speedup 1.0000x reference)
import functools

import jax
import jax.numpy as jnp
from jax import lax
from jax.experimental import pallas as pl
from jax.experimental.pallas import tpu as pltpu
from jax.experimental.pallas import tpu_sc as plsc

N_NODES = 10000
D_FEAT = 128
UNITS = 128

NC = 2
NS = 16
DQ = D_FEAT // NC
BATCH = 128
NBUF = 2
CH = 32
CHE = CH * BATCH
ACC_ROWS = 10240
ZROWS = 32
X_ROWS_PER_TILE = N_NODES // NS
OUT_ROWS_PER_TILE = ACC_ROWS // NS


def _sc_aggregate(edges_per_tile: int, full_tiles: int):
    nbatch = edges_per_tile // BATCH
    nchunk = nbatch // CH

    mesh = plsc.VectorSubcoreMesh(core_axis_name="c", subcore_axis_name="s")

    @functools.partial(
        pl.kernel,
        out_type=jax.ShapeDtypeStruct((NC, ACC_ROWS, DQ), jnp.float32),
        mesh=mesh,
        scratch_types=[
            pltpu.VMEM((2, CHE), jnp.int32),
            pltpu.VMEM((2, CH, BATCH), jnp.int32),
            pltpu.VMEM((NBUF, BATCH, DQ), jnp.float32),
            pltpu.VMEM((ZROWS, DQ), jnp.float32),
            pltpu.VMEM_SHARED((N_NODES, DQ), jnp.float32),
            pltpu.VMEM_SHARED((ACC_ROWS, DQ), jnp.float32),
        ] + [pltpu.SemaphoreType.DMA] * (2 * NBUF + 4),
        compiler_params=pltpu.CompilerParams(use_tc_tiling_on_sc=False),
    )
    def agg(src_hbm, dst_hbm, tsrc_hbm, tdst_hbm, xs_hbm, out_hbm,
            src_v, dst_v, rows_v, zbuf_v, x_sp, acc_sh, *sems):
        gsems = sems[:NBUF]
        ssems = sems[NBUF:2 * NBUF]
        isems = sems[2 * NBUF:2 * NBUF + 2]
        jsems = sems[2 * NBUF + 2:]
        c = lax.axis_index("c")
        s = lax.axis_index("s")

        ebase = s * edges_per_tile

        def idx_prefetch(ck):
            cb = ck % 2

            @pl.when(s < full_tiles)
            def _():
                pltpu.async_copy(src_hbm.at[pl.ds(ebase + ck * CHE, CHE)],
                                 src_v.at[cb], isems[cb])
                pltpu.async_copy(dst_hbm.at[pl.ds(s * nbatch + ck * CH, CH)],
                                 dst_v.at[cb], jsems[cb])

            @pl.when(s >= full_tiles)
            def _():
                st = s - full_tiles
                pltpu.async_copy(
                    tsrc_hbm.at[pl.ds(st * edges_per_tile + ck * CHE, CHE)],
                    src_v.at[cb], isems[cb])
                pltpu.async_copy(
                    tdst_hbm.at[pl.ds(st * nbatch + ck * CH, CH)],
                    dst_v.at[cb], jsems[cb])

        def idx_wait(ck):
            cb = ck % 2
            pltpu.make_async_copy(src_hbm.at[pl.ds(0, CHE)], src_v.at[cb],
                                  isems[cb]).wait()
            pltpu.make_async_copy(dst_hbm.at[pl.ds(0, CH)], dst_v.at[cb],
                                  jsems[cb]).wait()

        idx_prefetch(0)

        zvec = jnp.zeros((16,), jnp.float32)
        def zero_body(i, _):
            for j in range(DQ // 16):
                zbuf_v[i, pl.ds(j * 16, 16)] = zvec
            return 0
        lax.fori_loop(0, ZROWS, zero_body, 0)

        xbase = s * X_ROWS_PER_TILE
        pltpu.sync_copy(xs_hbm.at[pl.ds(xbase, X_ROWS_PER_TILE),
                                  pl.ds(c * DQ, DQ)],
                        x_sp.at[pl.ds(xbase, X_ROWS_PER_TILE)])
        nz = ACC_ROWS // NS // ZROWS
        for k in range(nz):
            pltpu.sync_copy(zbuf_v,
                            acc_sh.at[pl.ds((s * nz + k) * ZROWS, ZROWS)])
        plsc.subcore_barrier()

        for ck in range(nchunk):
            cb = ck % 2
            idx_wait(ck)
            if ck + 1 < nchunk:
                idx_prefetch(ck + 1)

            def gather_start(j, b):
                off = pl.multiple_of(j * BATCH, BATCH)
                pltpu.async_copy(x_sp.at[src_v.at[cb, pl.ds(off, BATCH)]],
                                 rows_v.at[b], gsems[b])

            def gather_wait(b):
                pltpu.make_async_copy(x_sp.at[src_v.at[cb, pl.ds(0, BATCH)]],
                                      rows_v.at[b], gsems[b]).wait()

            def scatter_start(j, b):
                pltpu.async_copy(rows_v.at[b], acc_sh.at[dst_v.at[cb, j]],
                                 ssems[b], add=True)

            def scatter_wait(b):
                pltpu.make_async_copy(rows_v.at[b], acc_sh.at[dst_v.at[cb, 0]],
                                      ssems[b]).wait()

            for b in range(NBUF - 1):
                gather_start(b, b)

            def pipe(t, _):
                j0 = t * NBUF
                for b in range(NBUF):
                    j = j0 + b
                    gather_wait(b)
                    scatter_start(j, b)
                    jn = j + NBUF - 1
                    bn = (b + NBUF - 1) % NBUF
                    @pl.when(jn < CH)
                    def _():
                        @pl.when(jn >= NBUF)
                        def _():
                            scatter_wait(bn)
                        gather_start(jn, bn)
                return 0

            lax.fori_loop(0, CH // NBUF, pipe, 0)
            for b in range(NBUF):
                scatter_wait(b)

        plsc.subcore_barrier()

        obase = s * OUT_ROWS_PER_TILE
        pltpu.sync_copy(acc_sh.at[pl.ds(obase, OUT_ROWS_PER_TILE)],
                        out_hbm.at[c, pl.ds(obase, OUT_ROWS_PER_TILE)])

    return agg


def _tc_body(p_ref, w_ref, b_ref, o_ref):
    w = w_ref[...]
    acc = b_ref[...].astype(jnp.float32)
    parts = []
    for ci in range(NC):
        parts.append(jnp.dot(p_ref[ci], w[ci * DQ:(ci + 1) * DQ],
                             preferred_element_type=jnp.float32))
    o_ref[...] = parts[0] + parts[1] + acc


def _tc_matmul(partials, w, b2d):
    rows = 2000
    grid = N_NODES // rows
    return pl.pallas_call(
        _tc_body,
        grid=(grid,),
        in_specs=[
            pl.BlockSpec((NC, rows, DQ), lambda i: (0, i, 0)),
            pl.BlockSpec((D_FEAT, UNITS), lambda i: (0, 0)),
            pl.BlockSpec((1, UNITS), lambda i: (0, 0)),
        ],
        out_specs=pl.BlockSpec((rows, UNITS), lambda i: (i, 0)),
        out_shape=jax.ShapeDtypeStruct((N_NODES, UNITS), jnp.float32),
    )(partials, w, b2d)


@jax.jit
def kernel(X, ref_A, ref_B, w, b):
    E = ref_A.shape[0]
    chunk = NS * CHE
    e_pad = ((E + chunk - 1) // chunk) * chunk
    ept = e_pad // NS
    full_tiles = min(E // ept, NS - 1)
    src = ref_A.astype(jnp.int32)
    dst = ref_B.astype(jnp.int32)
    tpad = NS * ept - E
    tsrc = jnp.concatenate(
        [src[full_tiles * ept:], jnp.zeros((tpad,), jnp.int32)])
    tdst = jnp.concatenate(
        [dst[full_tiles * ept:],
         jnp.full((tpad,), N_NODES, jnp.int32)])
    dst2d = dst[:full_tiles * ept].reshape(full_tiles * ept // BATCH, BATCH)
    tdst2d = tdst.reshape((NS - full_tiles) * ept // BATCH, BATCH)

    partials = _sc_aggregate(ept, full_tiles)(src, dst2d, tsrc, tdst2d, X)
    out = _tc_matmul(partials, w, b.reshape(1, UNITS))
    return out


# speedup vs baseline: 1.0359x; 1.0359x over previous
import functools

import jax
import jax.numpy as jnp
from jax import lax
from jax.experimental import pallas as pl
from jax.experimental.pallas import tpu as pltpu
from jax.experimental.pallas import tpu_sc as plsc

N_NODES = 10000
D_FEAT = 128
UNITS = 128

NC = 2
NS = 16
DQ = D_FEAT // NC
BATCH = 128
NBUF = 4
CH = 16
CHE = CH * BATCH
ACC_ROWS = 10240
ZROWS = 32
X_ROWS_PER_TILE = N_NODES // NS
OUT_ROWS_PER_TILE = ACC_ROWS // NS


def _sc_aggregate(edges_per_tile: int, full_tiles: int):
    nbatch = edges_per_tile // BATCH
    nchunk = nbatch // CH

    mesh = plsc.VectorSubcoreMesh(core_axis_name="c", subcore_axis_name="s")

    @functools.partial(
        pl.kernel,
        out_type=jax.ShapeDtypeStruct((NC, ACC_ROWS, DQ), jnp.float32),
        mesh=mesh,
        scratch_types=[
            pltpu.VMEM((2, CHE), jnp.int32),
            pltpu.VMEM((2, CH, BATCH), jnp.int32),
            pltpu.VMEM((NBUF, BATCH, DQ), jnp.float32),
            pltpu.VMEM((ZROWS, DQ), jnp.float32),
            pltpu.VMEM_SHARED((N_NODES, DQ), jnp.float32),
            pltpu.VMEM_SHARED((ACC_ROWS, DQ), jnp.float32),
        ] + [pltpu.SemaphoreType.DMA] * (2 * NBUF + 4),
        compiler_params=pltpu.CompilerParams(use_tc_tiling_on_sc=False),
    )
    def agg(src_hbm, dst_hbm, tsrc_hbm, tdst_hbm, xs_hbm, out_hbm,
            src_v, dst_v, rows_v, zbuf_v, x_sp, acc_sh, *sems):
        gsems = sems[:NBUF]
        ssems = sems[NBUF:2 * NBUF]
        isems = sems[2 * NBUF:2 * NBUF + 2]
        jsems = sems[2 * NBUF + 2:]
        c = lax.axis_index("c")
        s = lax.axis_index("s")

        ebase = s * edges_per_tile

        def idx_prefetch(ck):
            cb = ck % 2

            @pl.when(s < full_tiles)
            def _():
                pltpu.async_copy(src_hbm.at[pl.ds(ebase + ck * CHE, CHE)],
                                 src_v.at[cb], isems[cb])
                pltpu.async_copy(dst_hbm.at[pl.ds(s * nbatch + ck * CH, CH)],
                                 dst_v.at[cb], jsems[cb])

            @pl.when(s >= full_tiles)
            def _():
                st = s - full_tiles
                pltpu.async_copy(
                    tsrc_hbm.at[pl.ds(st * edges_per_tile + ck * CHE, CHE)],
                    src_v.at[cb], isems[cb])
                pltpu.async_copy(
                    tdst_hbm.at[pl.ds(st * nbatch + ck * CH, CH)],
                    dst_v.at[cb], jsems[cb])

        def idx_wait(ck):
            cb = ck % 2
            pltpu.make_async_copy(src_hbm.at[pl.ds(0, CHE)], src_v.at[cb],
                                  isems[cb]).wait()
            pltpu.make_async_copy(dst_hbm.at[pl.ds(0, CH)], dst_v.at[cb],
                                  jsems[cb]).wait()

        idx_prefetch(0)

        zvec = jnp.zeros((16,), jnp.float32)
        def zero_body(i, _):
            for j in range(DQ // 16):
                zbuf_v[i, pl.ds(j * 16, 16)] = zvec
            return 0
        lax.fori_loop(0, ZROWS, zero_body, 0)

        xbase = s * X_ROWS_PER_TILE
        pltpu.sync_copy(xs_hbm.at[pl.ds(xbase, X_ROWS_PER_TILE),
                                  pl.ds(c * DQ, DQ)],
                        x_sp.at[pl.ds(xbase, X_ROWS_PER_TILE)])
        nz = ACC_ROWS // NS // ZROWS
        for k in range(nz):
            pltpu.sync_copy(zbuf_v,
                            acc_sh.at[pl.ds((s * nz + k) * ZROWS, ZROWS)])
        plsc.subcore_barrier()

        for ck in range(nchunk):
            cb = ck % 2
            idx_wait(ck)
            if ck + 1 < nchunk:
                idx_prefetch(ck + 1)

            def gather_start(j, b):
                off = pl.multiple_of(j * BATCH, BATCH)
                pltpu.async_copy(x_sp.at[src_v.at[cb, pl.ds(off, BATCH)]],
                                 rows_v.at[b], gsems[b])

            def gather_wait(b):
                pltpu.make_async_copy(x_sp.at[src_v.at[cb, pl.ds(0, BATCH)]],
                                      rows_v.at[b], gsems[b]).wait()

            def scatter_start(j, b):
                pltpu.async_copy(rows_v.at[b], acc_sh.at[dst_v.at[cb, j]],
                                 ssems[b], add=True)

            def scatter_wait(b):
                pltpu.make_async_copy(rows_v.at[b], acc_sh.at[dst_v.at[cb, 0]],
                                      ssems[b]).wait()

            for b in range(NBUF - 1):
                gather_start(b, b)

            def pipe(t, _):
                j0 = t * NBUF
                for b in range(NBUF):
                    j = j0 + b
                    gather_wait(b)
                    scatter_start(j, b)
                    jn = j + NBUF - 1
                    bn = (b + NBUF - 1) % NBUF
                    @pl.when(jn < CH)
                    def _():
                        @pl.when(jn >= NBUF)
                        def _():
                            scatter_wait(bn)
                        gather_start(jn, bn)
                return 0

            lax.fori_loop(0, CH // NBUF, pipe, 0)
            for b in range(NBUF):
                scatter_wait(b)

        plsc.subcore_barrier()

        obase = s * OUT_ROWS_PER_TILE
        pltpu.sync_copy(acc_sh.at[pl.ds(obase, OUT_ROWS_PER_TILE)],
                        out_hbm.at[c, pl.ds(obase, OUT_ROWS_PER_TILE)])

    return agg


def _tc_body(p_ref, w_ref, b_ref, o_ref):
    w = w_ref[...]
    acc = b_ref[...].astype(jnp.float32)
    parts = []
    for ci in range(NC):
        parts.append(jnp.dot(p_ref[ci], w[ci * DQ:(ci + 1) * DQ],
                             preferred_element_type=jnp.float32))
    o_ref[...] = parts[0] + parts[1] + acc


def _tc_matmul(partials, w, b2d):
    rows = 2000
    grid = N_NODES // rows
    return pl.pallas_call(
        _tc_body,
        grid=(grid,),
        in_specs=[
            pl.BlockSpec((NC, rows, DQ), lambda i: (0, i, 0)),
            pl.BlockSpec((D_FEAT, UNITS), lambda i: (0, 0)),
            pl.BlockSpec((1, UNITS), lambda i: (0, 0)),
        ],
        out_specs=pl.BlockSpec((rows, UNITS), lambda i: (i, 0)),
        out_shape=jax.ShapeDtypeStruct((N_NODES, UNITS), jnp.float32),
    )(partials, w, b2d)


@jax.jit
def kernel(X, ref_A, ref_B, w, b):
    E = ref_A.shape[0]
    chunk = NS * CHE
    e_pad = ((E + chunk - 1) // chunk) * chunk
    ept = e_pad // NS
    full_tiles = min(E // ept, NS - 1)
    src = ref_A.astype(jnp.int32)
    dst = ref_B.astype(jnp.int32)
    tpad = NS * ept - E
    tsrc = jnp.concatenate(
        [src[full_tiles * ept:], jnp.zeros((tpad,), jnp.int32)])
    tdst = jnp.concatenate(
        [dst[full_tiles * ept:],
         jnp.full((tpad,), N_NODES, jnp.int32)])
    dst2d = dst[:full_tiles * ept].reshape(full_tiles * ept // BATCH, BATCH)
    tdst2d = tdst.reshape((NS - full_tiles) * ept // BATCH, BATCH)

    partials = _sc_aggregate(ept, full_tiles)(src, dst2d, tsrc, tdst2d, X)
    out = _tc_matmul(partials, w, b.reshape(1, UNITS))
    return out


# speedup vs baseline: 1.0772x; 1.0399x over previous
import functools

import jax
import jax.numpy as jnp
from jax import lax
from jax.experimental import pallas as pl
from jax.experimental.pallas import tpu as pltpu
from jax.experimental.pallas import tpu_sc as plsc

N_NODES = 10000
D_FEAT = 128
UNITS = 128

NC = 2
NS = 16
DQ = D_FEAT // NC
BATCH = 128
NBUF = 4
CH = 32
CHE = CH * BATCH
ACC_ROWS = 10240
ZROWS = 8
X_ROWS_PER_TILE = N_NODES // NS
OUT_ROWS_PER_TILE = ACC_ROWS // NS


def _sc_aggregate(edges_per_tile: int, full_tiles: int):
    nbatch = edges_per_tile // BATCH
    nchunk = nbatch // CH

    mesh = plsc.VectorSubcoreMesh(core_axis_name="c", subcore_axis_name="s")

    @functools.partial(
        pl.kernel,
        out_type=jax.ShapeDtypeStruct((NC, ACC_ROWS, DQ), jnp.float32),
        mesh=mesh,
        scratch_types=[
            pltpu.VMEM((2, CHE), jnp.int32),
            pltpu.VMEM((2, CH, BATCH), jnp.int32),
            pltpu.VMEM((NBUF, BATCH, DQ), jnp.float32),
            pltpu.VMEM((ZROWS, DQ), jnp.float32),
            pltpu.VMEM_SHARED((N_NODES, DQ), jnp.float32),
            pltpu.VMEM_SHARED((ACC_ROWS, DQ), jnp.float32),
        ] + [pltpu.SemaphoreType.DMA] * (2 * NBUF + 4),
        compiler_params=pltpu.CompilerParams(use_tc_tiling_on_sc=False),
    )
    def agg(src_hbm, dst_hbm, tsrc_hbm, tdst_hbm, xs_hbm, out_hbm,
            src_v, dst_v, rows_v, zbuf_v, x_sp, acc_sh, *sems):
        gsems = sems[:NBUF]
        ssems = sems[NBUF:2 * NBUF]
        isems = sems[2 * NBUF:2 * NBUF + 2]
        jsems = sems[2 * NBUF + 2:]
        c = lax.axis_index("c")
        s = lax.axis_index("s")

        ebase = s * edges_per_tile

        def idx_prefetch(ck):
            cb = ck % 2

            @pl.when(s < full_tiles)
            def _():
                pltpu.async_copy(src_hbm.at[pl.ds(ebase + ck * CHE, CHE)],
                                 src_v.at[cb], isems[cb])
                pltpu.async_copy(dst_hbm.at[pl.ds(s * nbatch + ck * CH, CH)],
                                 dst_v.at[cb], jsems[cb])

            @pl.when(s >= full_tiles)
            def _():
                st = s - full_tiles
                pltpu.async_copy(
                    tsrc_hbm.at[pl.ds(st * edges_per_tile + ck * CHE, CHE)],
                    src_v.at[cb], isems[cb])
                pltpu.async_copy(
                    tdst_hbm.at[pl.ds(st * nbatch + ck * CH, CH)],
                    dst_v.at[cb], jsems[cb])

        def idx_wait(ck):
            cb = ck % 2
            pltpu.make_async_copy(src_hbm.at[pl.ds(0, CHE)], src_v.at[cb],
                                  isems[cb]).wait()
            pltpu.make_async_copy(dst_hbm.at[pl.ds(0, CH)], dst_v.at[cb],
                                  jsems[cb]).wait()

        idx_prefetch(0)

        zvec = jnp.zeros((16,), jnp.float32)
        def zero_body(i, _):
            for j in range(DQ // 16):
                zbuf_v[i, pl.ds(j * 16, 16)] = zvec
            return 0
        lax.fori_loop(0, ZROWS, zero_body, 0)

        xbase = s * X_ROWS_PER_TILE
        pltpu.sync_copy(xs_hbm.at[pl.ds(xbase, X_ROWS_PER_TILE),
                                  pl.ds(c * DQ, DQ)],
                        x_sp.at[pl.ds(xbase, X_ROWS_PER_TILE)])
        nz = ACC_ROWS // NS // ZROWS
        for k in range(nz):
            pltpu.sync_copy(zbuf_v,
                            acc_sh.at[pl.ds((s * nz + k) * ZROWS, ZROWS)])
        plsc.subcore_barrier()

        for ck in range(nchunk):
            cb = ck % 2
            idx_wait(ck)
            if ck + 1 < nchunk:
                idx_prefetch(ck + 1)

            def gather_start(j, b):
                off = pl.multiple_of(j * BATCH, BATCH)
                pltpu.async_copy(x_sp.at[src_v.at[cb, pl.ds(off, BATCH)]],
                                 rows_v.at[b], gsems[b])

            def gather_wait(b):
                pltpu.make_async_copy(x_sp.at[src_v.at[cb, pl.ds(0, BATCH)]],
                                      rows_v.at[b], gsems[b]).wait()

            def scatter_start(j, b):
                pltpu.async_copy(rows_v.at[b], acc_sh.at[dst_v.at[cb, j]],
                                 ssems[b], add=True)

            def scatter_wait(b):
                pltpu.make_async_copy(rows_v.at[b], acc_sh.at[dst_v.at[cb, 0]],
                                      ssems[b]).wait()

            for b in range(NBUF - 1):
                gather_start(b, b)

            def pipe(t, _):
                j0 = t * NBUF
                for b in range(NBUF):
                    j = j0 + b
                    gather_wait(b)
                    scatter_start(j, b)
                    jn = j + NBUF - 1
                    bn = (b + NBUF - 1) % NBUF
                    @pl.when(jn < CH)
                    def _():
                        @pl.when(jn >= NBUF)
                        def _():
                            scatter_wait(bn)
                        gather_start(jn, bn)
                return 0

            lax.fori_loop(0, CH // NBUF, pipe, 0)
            for b in range(NBUF):
                scatter_wait(b)

        plsc.subcore_barrier()

        obase = s * OUT_ROWS_PER_TILE
        pltpu.sync_copy(acc_sh.at[pl.ds(obase, OUT_ROWS_PER_TILE)],
                        out_hbm.at[c, pl.ds(obase, OUT_ROWS_PER_TILE)])

    return agg


def _tc_body(p_ref, w_ref, b_ref, o_ref):
    w = w_ref[...]
    acc = b_ref[...].astype(jnp.float32)
    parts = []
    for ci in range(NC):
        parts.append(jnp.dot(p_ref[ci], w[ci * DQ:(ci + 1) * DQ],
                             preferred_element_type=jnp.float32))
    o_ref[...] = parts[0] + parts[1] + acc


def _tc_matmul(partials, w, b2d):
    rows = 2000
    grid = N_NODES // rows
    return pl.pallas_call(
        _tc_body,
        grid=(grid,),
        in_specs=[
            pl.BlockSpec((NC, rows, DQ), lambda i: (0, i, 0)),
            pl.BlockSpec((D_FEAT, UNITS), lambda i: (0, 0)),
            pl.BlockSpec((1, UNITS), lambda i: (0, 0)),
        ],
        out_specs=pl.BlockSpec((rows, UNITS), lambda i: (i, 0)),
        out_shape=jax.ShapeDtypeStruct((N_NODES, UNITS), jnp.float32),
    )(partials, w, b2d)


@jax.jit
def kernel(X, ref_A, ref_B, w, b):
    E = ref_A.shape[0]
    chunk = NS * CHE
    e_pad = ((E + chunk - 1) // chunk) * chunk
    ept = e_pad // NS
    full_tiles = min(E // ept, NS - 1)
    src = ref_A.astype(jnp.int32)
    dst = ref_B.astype(jnp.int32)
    tpad = NS * ept - E
    tsrc = jnp.concatenate(
        [src[full_tiles * ept:], jnp.zeros((tpad,), jnp.int32)])
    tdst = jnp.concatenate(
        [dst[full_tiles * ept:],
         jnp.full((tpad,), N_NODES, jnp.int32)])
    dst2d = dst[:full_tiles * ept].reshape(full_tiles * ept // BATCH, BATCH)
    tdst2d = tdst.reshape((NS - full_tiles) * ept // BATCH, BATCH)

    partials = _sc_aggregate(ept, full_tiles)(src, dst2d, tsrc, tdst2d, X)
    out = _tc_matmul(partials, w, b.reshape(1, UNITS))
    return out
